# baseline (speedup 1.0000x reference)
import jax
import jax.numpy as jnp
from jax.experimental import pallas as pl
from jax.experimental.pallas import tpu as pltpu

_N = 4096
_BC = 256
_NBLK = _N // _BC
_RC = 1024
_NCH = _N // _RC
_LRC = 2048
_NLC = _N // _LRC
_NIN = 8
_NOUT = 4
_K0 = _N // 3
_K1 = 2 * (_N // 3)
_MASK = 0x7FFFFFFF


def _kth_key(keys, k):
    n_neg = jnp.sum((keys < jnp.int32(0)).astype(jnp.int32))
    is_neg = jnp.int32(k + 1) <= n_neg
    lo0 = jnp.where(is_neg, jnp.int32(-(2 ** 31)), jnp.int32(0))
    hi0 = jnp.where(is_neg, jnp.int32(-1), jnp.int32(2 ** 31 - 1))

    def body(_, lohi):
        lo, hi = lohi
        mid = lo + (hi - lo) // 2
        cnt = jnp.sum((keys <= mid).astype(jnp.int32))
        ge = cnt >= jnp.int32(k + 1)
        return jnp.where(ge, lo, mid + 1), jnp.where(ge, mid, hi)

    lo, _ = jax.lax.fori_loop(0, 31, body, (lo0, hi0))
    return lo


def _load(w_ref, in_buf, in_sems, j, b, c):
    return pltpu.make_async_copy(
        w_ref.at[pl.ds(c * _LRC, _LRC), pl.ds(j * _BC, _BC)],
        in_buf.at[b, pl.ds(c * _LRC, _LRC), :],
        in_sems.at[b, c])


def _start_load(w_ref, in_buf, in_sems, j, b):
    for c in range(_NLC):
        _load(w_ref, in_buf, in_sems, j, b, c).start()


def _store(o_ref, out_buf, out_sems, j, b, r):
    return pltpu.make_async_copy(
        out_buf.at[b, pl.ds(r * _RC, _RC), :],
        o_ref.at[pl.ds(r * _RC, _RC), pl.ds(j * _BC, _BC)],
        out_sems.at[b, r])


def _body(scores8_ref, scores_full_ref, w_ref, o_ref,
          in_buf, out_buf, in_sems, out_sems):
    for jj in range(min(_NIN, _NBLK)):
        _start_load(w_ref, in_buf, in_sems, jj, jj)

    sf = scores_full_ref[...]
    bbits = jax.lax.bitcast_convert_type(sf, jnp.int32)
    keys = bbits ^ ((bbits >> 31) & jnp.int32(_MASK))

    def unmap(kk):
        return jax.lax.bitcast_convert_type(
            jnp.where(kk >= 0, kk, kk ^ jnp.int32(_MASK)), jnp.float32)

    t0 = unmap(_kth_key(keys, _K0))
    t1 = unmap(_kth_key(keys, _K1))

    def block(j, _):
        b = j % _NIN
        b2 = j % _NOUT
        _load(w_ref, in_buf, in_sems, j, b, 0).wait()
        w0 = in_buf[b, 0:_LRC, :]
        mn = jnp.min(w0, axis=0, keepdims=True)
        mx = jnp.max(w0, axis=0, keepdims=True)
        _load(w_ref, in_buf, in_sems, j, b, 1).wait()
        w1 = in_buf[b, _LRC:_N, :]
        mn = jnp.minimum(mn, jnp.min(w1, axis=0, keepdims=True))
        mx = jnp.maximum(mx, jnp.max(w1, axis=0, keepdims=True))

        s = scores8_ref[pl.ds(j, 1), :]
        half = jnp.where(s <= t0, 2.0, jnp.where(s <= t1, 8.0, 32.0))
        q_min = -half
        q_max = half - 1.0
        scale = (mx - mn) / (q_max - q_min)
        scale = jnp.where(jnp.abs(scale) < 1e-6, jnp.float32(1e-6), scale)
        inv = 1.0 / scale
        zp = jnp.clip(jnp.round(q_min - mn / scale), q_min, q_max)

        @pl.when(j >= _NOUT)
        def _():
            for r in range(_NCH):
                _store(o_ref, out_buf, out_sems, j - _NOUT, b2, r).wait()

        w = in_buf[b]
        for r in range(_NCH):
            wc = w[r * _RC:(r + 1) * _RC, :]
            q = jnp.clip(jnp.round(wc * inv) + zp, -128.0, 127.0)
            out_buf[b2, r * _RC:(r + 1) * _RC, :] = (q - zp) * scale
            _store(o_ref, out_buf, out_sems, j, b2, r).start()

        @pl.when(j + _NIN < _NBLK)
        def _():
            _start_load(w_ref, in_buf, in_sems, j + _NIN, b)

        return 0

    jax.lax.fori_loop(0, _NBLK, block, 0)
    for jj in range(max(0, _NBLK - _NOUT), _NBLK):
        for r in range(_NCH):
            _store(o_ref, out_buf, out_sems, jj, jj % _NOUT, r).wait()


def kernel(weight, scores):
    scores8 = scores.reshape(_NBLK, _BC)
    scores_full = scores.reshape(32, 128)
    return pl.pallas_call(
        _body,
        in_specs=[
            pl.BlockSpec(memory_space=pltpu.VMEM),
            pl.BlockSpec(memory_space=pltpu.VMEM),
            pl.BlockSpec(memory_space=pl.ANY),
        ],
        out_specs=pl.BlockSpec(memory_space=pl.ANY),
        out_shape=jax.ShapeDtypeStruct((_N, _N), jnp.float32),
        scratch_shapes=[
            pltpu.VMEM((_NIN, _N, _BC), jnp.float32),
            pltpu.VMEM((_NOUT, _N, _BC), jnp.float32),
            pltpu.SemaphoreType.DMA((_NIN, _NLC)),
            pltpu.SemaphoreType.DMA((_NOUT, _NCH)),
        ],
    )(scores8, scores_full, weight)


# speedup vs baseline: 1.0039x; 1.0039x over previous
import jax
import jax.numpy as jnp
from jax.experimental import pallas as pl
from jax.experimental.pallas import tpu as pltpu

_N = 4096
_BC = 512
_NBLK = _N // _BC
_RC = 512
_NCH = _N // _RC
_LRC = 2048
_NLC = _N // _LRC
_NIN = 5
_NOUT = 2
_K0 = _N // 3
_K1 = 2 * (_N // 3)
_MASK = 0x7FFFFFFF


def _kth_key(keys, k):
    n_neg = jnp.sum((keys < jnp.int32(0)).astype(jnp.int32))
    is_neg = jnp.int32(k + 1) <= n_neg
    lo0 = jnp.where(is_neg, jnp.int32(-(2 ** 31)), jnp.int32(0))
    hi0 = jnp.where(is_neg, jnp.int32(-1), jnp.int32(2 ** 31 - 1))

    def body(_, lohi):
        lo, hi = lohi
        mid = lo + (hi - lo) // 2
        cnt = jnp.sum((keys <= mid).astype(jnp.int32))
        ge = cnt >= jnp.int32(k + 1)
        return jnp.where(ge, lo, mid + 1), jnp.where(ge, mid, hi)

    lo, _ = jax.lax.fori_loop(0, 31, body, (lo0, hi0))
    return lo


def _load(w_ref, in_buf, in_sems, j, b, c):
    return pltpu.make_async_copy(
        w_ref.at[pl.ds(c * _LRC, _LRC), pl.ds(j * _BC, _BC)],
        in_buf.at[b, pl.ds(c * _LRC, _LRC), :],
        in_sems.at[b, c])


def _start_load(w_ref, in_buf, in_sems, j, b):
    for c in range(_NLC):
        _load(w_ref, in_buf, in_sems, j, b, c).start()


def _store(o_ref, out_buf, out_sems, j, b, r):
    return pltpu.make_async_copy(
        out_buf.at[b, pl.ds(r * _RC, _RC), :],
        o_ref.at[pl.ds(r * _RC, _RC), pl.ds(j * _BC, _BC)],
        out_sems.at[b, r])


def _body(scores8_ref, scores_full_ref, w_ref, o_ref,
          in_buf, out_buf, in_sems, out_sems):
    for jj in range(min(_NIN, _NBLK)):
        _start_load(w_ref, in_buf, in_sems, jj, jj)

    sf = scores_full_ref[...]
    bbits = jax.lax.bitcast_convert_type(sf, jnp.int32)
    keys = bbits ^ ((bbits >> 31) & jnp.int32(_MASK))

    def unmap(kk):
        return jax.lax.bitcast_convert_type(
            jnp.where(kk >= 0, kk, kk ^ jnp.int32(_MASK)), jnp.float32)

    t0 = unmap(_kth_key(keys, _K0))
    t1 = unmap(_kth_key(keys, _K1))

    def block(j, _):
        b = j % _NIN
        b2 = j % _NOUT
        _load(w_ref, in_buf, in_sems, j, b, 0).wait()
        w0 = in_buf[b, 0:_LRC, :]
        mn = jnp.min(w0, axis=0, keepdims=True)
        mx = jnp.max(w0, axis=0, keepdims=True)
        _load(w_ref, in_buf, in_sems, j, b, 1).wait()
        w1 = in_buf[b, _LRC:_N, :]
        mn = jnp.minimum(mn, jnp.min(w1, axis=0, keepdims=True))
        mx = jnp.maximum(mx, jnp.max(w1, axis=0, keepdims=True))

        s = scores8_ref[pl.ds(j, 1), :]
        half = jnp.where(s <= t0, 2.0, jnp.where(s <= t1, 8.0, 32.0))
        q_min = -half
        q_max = half - 1.0
        scale = (mx - mn) / (q_max - q_min)
        scale = jnp.where(jnp.abs(scale) < 1e-6, jnp.float32(1e-6), scale)
        inv = 1.0 / scale
        zp = jnp.clip(jnp.round(q_min - mn / scale), q_min, q_max)

        @pl.when(j >= _NOUT)
        def _():
            for r in range(_NCH):
                _store(o_ref, out_buf, out_sems, j - _NOUT, b2, r).wait()

        w = in_buf[b]
        for r in range(_NCH):
            wc = w[r * _RC:(r + 1) * _RC, :]
            q = jnp.clip(jnp.round(wc * inv) + zp, -128.0, 127.0)
            out_buf[b2, r * _RC:(r + 1) * _RC, :] = (q - zp) * scale
            _store(o_ref, out_buf, out_sems, j, b2, r).start()

        @pl.when(j + _NIN < _NBLK)
        def _():
            _start_load(w_ref, in_buf, in_sems, j + _NIN, b)

        return 0

    jax.lax.fori_loop(0, _NBLK, block, 0)
    for jj in range(max(0, _NBLK - _NOUT), _NBLK):
        for r in range(_NCH):
            _store(o_ref, out_buf, out_sems, jj, jj % _NOUT, r).wait()


def kernel(weight, scores):
    scores8 = scores.reshape(_NBLK, _BC)
    scores_full = scores.reshape(32, 128)
    return pl.pallas_call(
        _body,
        in_specs=[
            pl.BlockSpec(memory_space=pltpu.VMEM),
            pl.BlockSpec(memory_space=pltpu.VMEM),
            pl.BlockSpec(memory_space=pl.ANY),
        ],
        out_specs=pl.BlockSpec(memory_space=pl.ANY),
        out_shape=jax.ShapeDtypeStruct((_N, _N), jnp.float32),
        scratch_shapes=[
            pltpu.VMEM((_NIN, _N, _BC), jnp.float32),
            pltpu.VMEM((_NOUT, _N, _BC), jnp.float32),
            pltpu.SemaphoreType.DMA((_NIN, _NLC)),
            pltpu.SemaphoreType.DMA((_NOUT, _NCH)),
        ],
    )(scores8, scores_full, weight)
